# baseline (speedup 1.0000x reference)
import functools

import jax
import jax.numpy as jnp
from jax import lax
from jax.experimental import pallas as pl
from jax.experimental.pallas import tpu as pltpu
from jax.experimental.pallas import tpu_sc as plsc

_B = 16384
_D = 128
_NC = 2
_NS = 16
_NW = _NC * _NS
_BPW = _B // _NW
_CH = 128
_NCHUNK = _BPW // _CH

_mesh = plsc.VectorSubcoreMesh(core_axis_name="c", subcore_axis_name="s")


@functools.partial(
    pl.kernel,
    mesh=_mesh,
    out_type=jax.ShapeDtypeStruct((_NW * _NCHUNK, _CH, _D), jnp.float32),
    scratch_types=[
        pltpu.VMEM((_NCHUNK, _CH), jnp.int32),
        pltpu.VMEM((_NCHUNK, _CH, _D), jnp.float32),
        pltpu.SemaphoreType.DMA((_NCHUNK,)),
        pltpu.SemaphoreType.DMA,
    ],
)
def _gather_kernel(t_hbm, table_hbm, out_hbm, idx_v, rows_v, gsems, ssem):
    wid = lax.axis_index("s") * _NC + lax.axis_index("c")
    base = wid * _NCHUNK
    pltpu.sync_copy(t_hbm.at[pl.ds(base, _NCHUNK)], idx_v)
    gathers = [
        pltpu.async_copy(table_hbm.at[idx_v.at[j]], rows_v.at[j], gsems.at[j])
        for j in range(_NCHUNK)
    ]
    stores = []
    for j in range(_NCHUNK):
        gathers[j].wait()
        stores.append(pltpu.async_copy(rows_v.at[j], out_hbm.at[base + j], ssem))
    for s in stores:
        s.wait()


def kernel(t, embed_weight):
    t32 = t.astype(jnp.int32).reshape(_NW * _NCHUNK, _CH)
    out = _gather_kernel(t32, embed_weight)
    return out.reshape(_B, _D)


# speedup vs baseline: 1.1525x; 1.1525x over previous
import functools

import jax
import jax.numpy as jnp
from jax import lax
from jax.experimental import pallas as pl
from jax.experimental.pallas import tpu as pltpu
from jax.experimental.pallas import tpu_sc as plsc

_B = 16384
_D = 128
_NC = 2
_NS = 16
_NW = _NC * _NS
_BPW = _B // _NW
_CH = 128
_NCHUNK = _BPW // _CH

_mesh = plsc.VectorSubcoreMesh(core_axis_name="c", subcore_axis_name="s")


@functools.partial(
    pl.kernel,
    mesh=_mesh,
    out_type=jax.ShapeDtypeStruct((_NW * _NCHUNK, _CH, _D), jnp.float32),
    scratch_types=[
        pltpu.VMEM((_NCHUNK, _CH), jnp.int32),
        pltpu.VMEM((_NCHUNK, _CH, _D), jnp.float32),
        pltpu.VMEM_SHARED((1000, _D), jnp.float32),
        pltpu.SemaphoreType.DMA((_NCHUNK,)),
        pltpu.SemaphoreType.DMA,
    ],
)
def _gather_kernel(t_hbm, table_hbm, out_hbm, idx_v, rows_v, table_sh, gsems, ssem):
    cid = lax.axis_index("c")
    sid = lax.axis_index("s")
    wid = sid * _NC + cid
    base = wid * _NCHUNK
    @pl.when(sid < 5)
    def _():
        pltpu.sync_copy(
            table_hbm.at[pl.ds(sid * 200, 200)], table_sh.at[pl.ds(sid * 200, 200)]
        )
    pltpu.sync_copy(t_hbm.at[pl.ds(base, _NCHUNK)], idx_v)
    plsc.subcore_barrier()
    gathers = [
        pltpu.async_copy(table_sh.at[idx_v.at[j]], rows_v.at[j], gsems.at[j])
        for j in range(_NCHUNK)
    ]
    stores = []
    for j in range(_NCHUNK):
        gathers[j].wait()
        stores.append(pltpu.async_copy(rows_v.at[j], out_hbm.at[base + j], ssem))
    for s in stores:
        s.wait()


def kernel(t, embed_weight):
    t32 = t.astype(jnp.int32).reshape(_NW * _NCHUNK, _CH)
    out = _gather_kernel(t32, embed_weight)
    return out.reshape(_B, _D)


# speedup vs baseline: 1.1702x; 1.0154x over previous
import functools

import jax
import jax.numpy as jnp
from jax import lax
from jax.experimental import pallas as pl
from jax.experimental.pallas import tpu as pltpu
from jax.experimental.pallas import tpu_sc as plsc

_B = 16384
_D = 128
_NC = 2
_NS = 16
_NW = _NC * _NS
_BPW = _B // _NW
_CH = 64
_NCHUNK = _BPW // _CH

_mesh = plsc.VectorSubcoreMesh(core_axis_name="c", subcore_axis_name="s")


@functools.partial(
    pl.kernel,
    mesh=_mesh,
    out_type=jax.ShapeDtypeStruct((_NW * _NCHUNK, _CH, _D), jnp.float32),
    scratch_types=[
        pltpu.VMEM((_NCHUNK, _CH), jnp.int32),
        pltpu.VMEM((_NCHUNK, _CH, _D), jnp.float32),
        pltpu.VMEM_SHARED((1000, _D), jnp.float32),
        pltpu.SemaphoreType.DMA((_NCHUNK,)),
        pltpu.SemaphoreType.DMA,
    ],
)
def _gather_kernel(t_hbm, table_hbm, out_hbm, idx_v, rows_v, table_sh, gsems, ssem):
    cid = lax.axis_index("c")
    sid = lax.axis_index("s")
    wid = sid * _NC + cid
    base = wid * _NCHUNK
    pltpu.sync_copy(t_hbm.at[pl.ds(base, _NCHUNK)], idx_v)
    @pl.when(sid < 15)
    def _():
        off = pl.multiple_of(sid * 64, 8)
        pltpu.sync_copy(table_hbm.at[pl.ds(off, 64)], table_sh.at[pl.ds(off, 64)])

    @pl.when(sid == 15)
    def _():
        pltpu.sync_copy(table_hbm.at[pl.ds(960, 40)], table_sh.at[pl.ds(960, 40)])

    plsc.subcore_barrier()
    gathers = [
        pltpu.async_copy(table_sh.at[idx_v.at[j]], rows_v.at[j], gsems.at[j])
        for j in range(_NCHUNK)
    ]
    stores = []
    for j in range(_NCHUNK):
        gathers[j].wait()
        stores.append(pltpu.async_copy(rows_v.at[j], out_hbm.at[base + j], ssem))
    for s in stores:
        s.wait()


def kernel(t, embed_weight):
    t32 = t.astype(jnp.int32).reshape(_NW * _NCHUNK, _CH)
    out = _gather_kernel(t32, embed_weight)
    return out.reshape(_B, _D)
